# baseline (speedup 1.0000x reference)
import jax
import jax.numpy as jnp
from jax.experimental import pallas as pl


def _dg(a, b):
    return jax.lax.dot_general(
        a, b, (((1,), (0,)), ((), ())), preferred_element_type=jnp.float32)


def kernel(x, edge_index_g, edge_index_g1, W_gat, attn_l, attn_r, W_sage,
           b_sage, W_hw, b_hw):
    f32 = jnp.float32
    N, D = x.shape
    E = edge_index_g.shape[1]
    H = attn_l.shape[0]
    HD = H * D
    EB = 256
    NBE = E // EB
    NBN = 2000
    GN = N // NBN

    eg = edge_index_g.astype(jnp.int32)
    eg1 = edge_index_g1.astype(jnp.int32)
    src_g = eg[0].reshape(NBE, 1, EB)
    dst_g = eg[1].reshape(NBE, 1, EB)
    src_s = eg1[0].reshape(NBE, 1, EB)
    dst_s = eg1[1].reshape(NBE, 1, EB)

    eye = jnp.eye(H, dtype=f32)
    A_l = (attn_l[:, :, None] * eye[:, None, :]).reshape(HD, H)
    A_r = (attn_r[:, :, None] * eye[:, None, :]).reshape(HD, H)
    b_sage2 = b_sage.reshape(1, D)
    b_hw2 = b_hw.reshape(1, 1)

    idx_spec = pl.BlockSpec((1, 1, EB), lambda i, *_: (i, 0, 0))
    idx_spec4 = pl.BlockSpec((1, 1, EB), lambda h, i: (i, 0, 0))

    def k1(x_ref, wg_ref, al_ref, ar_ref, f0_ref, f1_ref, el_ref, er_ref):
        xb = x_ref[...]
        f0 = _dg(xb, wg_ref[:, 0:D])
        f1 = _dg(xb, wg_ref[:, D:2 * D])
        f0_ref[...] = f0
        f1_ref[...] = f1
        feat = jnp.concatenate([f0, f1], axis=1)
        el_ref[...] = _dg(feat, al_ref[...])
        er_ref[...] = _dg(feat, ar_ref[...])

    feat0, feat1, el, er = pl.pallas_call(
        k1,
        grid=(GN,),
        in_specs=[
            pl.BlockSpec((NBN, D), lambda i: (i, 0)),
            pl.BlockSpec((D, HD), lambda i: (0, 0)),
            pl.BlockSpec((HD, H), lambda i: (0, 0)),
            pl.BlockSpec((HD, H), lambda i: (0, 0)),
        ],
        out_specs=[
            pl.BlockSpec((NBN, D), lambda i: (i, 0)),
            pl.BlockSpec((NBN, D), lambda i: (i, 0)),
            pl.BlockSpec((NBN, H), lambda i: (i, 0)),
            pl.BlockSpec((NBN, H), lambda i: (i, 0)),
        ],
        out_shape=[
            jax.ShapeDtypeStruct((N, D), f32),
            jax.ShapeDtypeStruct((N, D), f32),
            jax.ShapeDtypeStruct((N, H), f32),
            jax.ShapeDtypeStruct((N, H), f32),
        ],
    )(x, W_gat, A_l, A_r)

    neg_inf = float('-inf')

    def k2(src_ref, dst_ref, el_ref, er_ref, e_ref, emax_ref):
        i = pl.program_id(0)
        src = src_ref[0, 0, :]
        dst = dst_ref[0, 0, :]
        iota = jax.lax.broadcasted_iota(jnp.int32, (EB, N), 1)
        ms = (src[:, None] == iota)
        md = (dst[:, None] == iota)
        el_s = _dg(ms.astype(f32), el_ref[...])
        er_d = _dg(md.astype(f32), er_ref[...])
        z = el_s + er_d
        e = jnp.where(z >= 0, z, 0.2 * z)
        e_ref[...] = e
        rows = [jnp.max(jnp.where(md, e[:, h:h + 1], neg_inf), axis=0,
                        keepdims=True) for h in range(H)]
        cur = jnp.concatenate(rows, axis=0)

        @pl.when(i == 0)
        def _():
            emax_ref[...] = jnp.full((H, N), neg_inf, f32)

        emax_ref[...] = jnp.maximum(emax_ref[...], cur)

    e_arr, emax = pl.pallas_call(
        k2,
        grid=(NBE,),
        in_specs=[
            idx_spec, idx_spec,
            pl.BlockSpec((N, H), lambda i: (0, 0)),
            pl.BlockSpec((N, H), lambda i: (0, 0)),
        ],
        out_specs=[
            pl.BlockSpec((EB, H), lambda i: (i, 0)),
            pl.BlockSpec((H, N), lambda i: (0, 0)),
        ],
        out_shape=[
            jax.ShapeDtypeStruct((E, H), f32),
            jax.ShapeDtypeStruct((H, N), f32),
        ],
    )(src_g, dst_g, el, er)

    def k3(dst_ref, e_ref, emax_ref, a_ref, den_ref):
        i = pl.program_id(0)
        dst = dst_ref[0, 0, :]
        iota = jax.lax.broadcasted_iota(jnp.int32, (EB, N), 1)
        md = (dst[:, None] == iota).astype(f32)
        iota_t = jax.lax.broadcasted_iota(jnp.int32, (N, EB), 0)
        mdt = (dst[None, :] == iota_t).astype(f32)
        em = emax_ref[...]
        em = jnp.where(jnp.isfinite(em), em, 0.0)
        em_d = jnp.concatenate(
            [jnp.sum(md * em[h:h + 1, :], axis=1, keepdims=True)
             for h in range(H)], axis=1)
        a = jnp.exp(e_ref[...] - em_d)
        a_ref[...] = a

        @pl.when(i == 0)
        def _():
            den_ref[...] = jnp.zeros((N, H), f32)

        den_ref[...] += _dg(mdt, a)

    a_arr, denom = pl.pallas_call(
        k3,
        grid=(NBE,),
        in_specs=[
            idx_spec,
            pl.BlockSpec((EB, H), lambda i: (i, 0)),
            pl.BlockSpec((H, N), lambda i: (0, 0)),
        ],
        out_specs=[
            pl.BlockSpec((EB, H), lambda i: (i, 0)),
            pl.BlockSpec((N, H), lambda i: (0, 0)),
        ],
        out_shape=[
            jax.ShapeDtypeStruct((E, H), f32),
            jax.ShapeDtypeStruct((N, H), f32),
        ],
    )(dst_g, e_arr, emax)

    def make_k4(h):
        def k4(src_ref, dst_ref, a_ref, den_ref, feat_ref, out_ref):
            i = pl.program_id(0)
            src = src_ref[0, 0, :]
            dst = dst_ref[0, 0, :]
            iota = jax.lax.broadcasted_iota(jnp.int32, (EB, N), 1)
            ms = (src[:, None] == iota).astype(f32)
            md = (dst[:, None] == iota).astype(f32)
            iota_t = jax.lax.broadcasted_iota(jnp.int32, (N, EB), 0)
            mdt = (dst[None, :] == iota_t).astype(f32)
            den_d = _dg(md, den_ref[...])
            alpha = a_ref[...] / jnp.maximum(den_d, 1e-9)
            fs = _dg(ms, feat_ref[...])
            msg = fs * alpha[:, h:h + 1]

            @pl.when(i == 0)
            def _():
                out_ref[...] = jnp.zeros((N, D), f32)

            out_ref[...] += _dg(mdt, msg)
        return k4

    gat_heads = []
    for hh, feat_h in enumerate((feat0, feat1)):
        gat_heads.append(pl.pallas_call(
            make_k4(hh),
            grid=(NBE,),
            in_specs=[
                idx_spec, idx_spec,
                pl.BlockSpec((EB, H), lambda i: (i, 0)),
                pl.BlockSpec((N, H), lambda i: (0, 0)),
                pl.BlockSpec((N, D), lambda i: (0, 0)),
            ],
            out_specs=pl.BlockSpec((N, D), lambda i: (0, 0)),
            out_shape=jax.ShapeDtypeStruct((N, D), f32),
        )(src_g, dst_g, a_arr, denom, feat_h))
    gat0, gat1 = gat_heads

    def k5(src_ref, dst_ref, x_ref, agg_ref, deg_ref):
        i = pl.program_id(0)
        src = src_ref[0, 0, :]
        dst = dst_ref[0, 0, :]
        iota = jax.lax.broadcasted_iota(jnp.int32, (EB, N), 1)
        ms = (src[:, None] == iota).astype(f32)
        iota_t = jax.lax.broadcasted_iota(jnp.int32, (N, EB), 0)
        mdt = (dst[None, :] == iota_t).astype(f32)
        g = _dg(ms, x_ref[...])

        @pl.when(i == 0)
        def _():
            agg_ref[...] = jnp.zeros((N, D), f32)
            deg_ref[...] = jnp.zeros((N, 1), f32)

        agg_ref[...] += _dg(mdt, g)
        deg_ref[...] += _dg(mdt, jnp.ones((EB, 1), f32))

    agg, deg = pl.pallas_call(
        k5,
        grid=(NBE,),
        in_specs=[
            idx_spec, idx_spec,
            pl.BlockSpec((N, D), lambda i: (0, 0)),
        ],
        out_specs=[
            pl.BlockSpec((N, D), lambda i: (0, 0)),
            pl.BlockSpec((N, 1), lambda i: (0, 0)),
        ],
        out_shape=[
            jax.ShapeDtypeStruct((N, D), f32),
            jax.ShapeDtypeStruct((N, 1), f32),
        ],
    )(src_s, dst_s, x)

    def k6(x_ref, g0_ref, g1_ref, agg_ref, deg_ref, ws_ref, bs_ref, wh_ref,
           bh_ref, out_ref):
        xb = x_ref[...]
        hh = (agg_ref[...] + xb) / (deg_ref[...] + 1.0)
        sage = _dg(hh, ws_ref[...]) + bs_ref[...]
        wh = wh_ref[...]
        bh = bh_ref[...]
        acc = jnp.zeros_like(xb)
        for s in (g0_ref[...], g1_ref[...], sage):
            w = jax.nn.sigmoid(_dg(s, wh) + bh)
            acc = acc + s * w
        out_ref[...] = xb + acc

    emb = pl.pallas_call(
        k6,
        grid=(GN,),
        in_specs=[
            pl.BlockSpec((NBN, D), lambda i: (i, 0)),
            pl.BlockSpec((NBN, D), lambda i: (i, 0)),
            pl.BlockSpec((NBN, D), lambda i: (i, 0)),
            pl.BlockSpec((NBN, D), lambda i: (i, 0)),
            pl.BlockSpec((NBN, 1), lambda i: (i, 0)),
            pl.BlockSpec((D, D), lambda i: (0, 0)),
            pl.BlockSpec((1, D), lambda i: (0, 0)),
            pl.BlockSpec((D, 1), lambda i: (0, 0)),
            pl.BlockSpec((1, 1), lambda i: (0, 0)),
        ],
        out_specs=pl.BlockSpec((NBN, D), lambda i: (i, 0)),
        out_shape=jax.ShapeDtypeStruct((N, D), f32),
    )(x, gat0, gat1, agg, deg, W_sage, b_sage2, W_hw, b_hw2)

    return emb
